# Initial kernel scaffold; baseline (speedup 1.0000x reference)
import jax
import jax.numpy as jnp
from jax.experimental import pallas as pl


def kernel(z, edge_index, W1, b1, Wg, bg, W2, b2):
    raise NotImplementedError("write your pallas kernel here")


# speedup vs baseline: 14.4892x; 14.4892x over previous
import functools

import jax
import jax.numpy as jnp
from jax import lax
from jax.experimental import pallas as pl
from jax.experimental.pallas import tpu as pltpu
from jax.experimental.pallas import tpu_sc as plsc

_N = 10000
_D = 128
_E = 320000
_NC = 2
_NS = 16
_NW = _NC * _NS
_CHUNK = 128
_NPAD = 10240
_ROWS_PER_SUB = _NPAD // _NS
_EP = ((_E + _NW * _CHUNK - 1) // (_NW * _CHUNK)) * _NW * _CHUNK
_EDGES_PER_W = _EP // _NW
_CHUNKS_PER_W = _EDGES_PER_W // _CHUNK
_BLK = 256


def _sc_degree(dst_pad):
    mesh = plsc.VectorSubcoreMesh(core_axis_name="c", subcore_axis_name="s")

    @functools.partial(
        pl.kernel,
        out_type=jax.ShapeDtypeStruct((_NC, _NPAD), jnp.float32),
        mesh=mesh,
        scratch_types=[
            pltpu.VMEM((_CHUNK,), jnp.int32),
            pltpu.VMEM((_CHUNK,), jnp.float32),
            pltpu.VMEM((_ROWS_PER_SUB,), jnp.float32),
            pltpu.VMEM_SHARED((_NPAD,), jnp.float32),
        ],
    )
    def k(dst_hbm, deg_hbm, idx_v, ones_v, zeros_v, deg_sh):
        c = lax.axis_index("c")
        s = lax.axis_index("s")
        w = c * _NS + s

        def fill_ones(i, _):
            ones_v[pl.ds(i * 16, 16)] = jnp.ones((16,), jnp.float32)
            return 0

        lax.fori_loop(0, _CHUNK // 16, fill_ones, 0)

        def fill_zeros(i, _):
            zeros_v[pl.ds(i * 16, 16)] = jnp.zeros((16,), jnp.float32)
            return 0

        lax.fori_loop(0, _ROWS_PER_SUB // 16, fill_zeros, 0)
        pltpu.sync_copy(zeros_v, deg_sh.at[pl.ds(s * _ROWS_PER_SUB, _ROWS_PER_SUB)])
        plsc.subcore_barrier()

        def body(ci, _):
            base = w * _EDGES_PER_W + ci * _CHUNK
            pltpu.sync_copy(dst_hbm.at[pl.ds(base, _CHUNK)], idx_v)
            pltpu.sync_copy(ones_v, deg_sh.at[idx_v], add=True)
            return 0

        lax.fori_loop(0, _CHUNKS_PER_W, body, 0)
        plsc.subcore_barrier()
        pltpu.sync_copy(
            deg_sh.at[pl.ds(s * _ROWS_PER_SUB, _ROWS_PER_SUB)],
            deg_hbm.at[c, pl.ds(s * _ROWS_PER_SUB, _ROWS_PER_SUB)],
        )

    return k(dst_pad)


def _sc_gather_scatter(g, src_pad, dst_pad):
    mesh = plsc.VectorSubcoreMesh(core_axis_name="c", subcore_axis_name="s")

    @functools.partial(
        pl.kernel,
        out_type=jax.ShapeDtypeStruct((_NC, _NPAD, _D), jnp.float32),
        mesh=mesh,
        scratch_types=[
            pltpu.VMEM((_CHUNK,), jnp.int32),
            pltpu.VMEM((_CHUNK,), jnp.int32),
            pltpu.VMEM((_CHUNK, _D), jnp.float32),
            pltpu.VMEM_SHARED((_NPAD, _D), jnp.float32),
            pltpu.SemaphoreType.DMA,
        ],
    )
    def k(g_hbm, src_hbm, dst_hbm, acc_hbm, sidx, didx, rows, acc_sh, sem):
        c = lax.axis_index("c")
        s = lax.axis_index("s")
        w = c * _NS + s

        def zero_row(r, _):
            for j in range(_D // 16):
                rows[r, pl.ds(j * 16, 16)] = jnp.zeros((16,), jnp.float32)
            return 0

        lax.fori_loop(0, _CHUNK, zero_row, 0)
        for t in range(_ROWS_PER_SUB // _CHUNK):
            pltpu.sync_copy(
                rows, acc_sh.at[pl.ds(s * _ROWS_PER_SUB + t * _CHUNK, _CHUNK)]
            )
        plsc.subcore_barrier()

        def body(ci, _):
            base = w * _EDGES_PER_W + ci * _CHUNK
            pltpu.sync_copy(src_hbm.at[pl.ds(base, _CHUNK)], sidx)
            pltpu.sync_copy(dst_hbm.at[pl.ds(base, _CHUNK)], didx)
            pltpu.async_copy(g_hbm.at[sidx], rows, sem).wait()
            pltpu.sync_copy(rows, acc_sh.at[didx], add=True)
            return 0

        lax.fori_loop(0, _CHUNKS_PER_W, body, 0)
        plsc.subcore_barrier()
        pltpu.sync_copy(
            acc_sh.at[pl.ds(s * _ROWS_PER_SUB, _ROWS_PER_SUB)],
            acc_hbm.at[c, pl.ds(s * _ROWS_PER_SUB, _ROWS_PER_SUB)],
        )

    return k(g, src_pad, dst_pad)


def _tc_pre(z_pad, W1, b1r, Wg, deg):
    def body(z_ref, w1_ref, b1_ref, wg_ref, deg_ref, g_ref):
        i = pl.program_id(0)
        h = jnp.maximum(
            jnp.dot(z_ref[...], w1_ref[...], preferred_element_type=jnp.float32)
            + b1_ref[...],
            0.0,
        )
        h2 = jnp.dot(h, wg_ref[...], preferred_element_type=jnp.float32)
        dsum = (
            deg_ref[0, pl.ds(i * _BLK, _BLK)]
            + deg_ref[1, pl.ds(i * _BLK, _BLK)]
            + 1.0
        )
        dinv = lax.rsqrt(dsum)
        g_ref[...] = h2 * dinv[:, None]

    return pl.pallas_call(
        body,
        grid=(_NPAD // _BLK,),
        in_specs=[
            pl.BlockSpec((_BLK, _D), lambda i: (i, 0)),
            pl.BlockSpec((_D, _D), lambda i: (0, 0)),
            pl.BlockSpec((1, _D), lambda i: (0, 0)),
            pl.BlockSpec((_D, _D), lambda i: (0, 0)),
            pl.BlockSpec((_NC, _NPAD), lambda i: (0, 0)),
        ],
        out_specs=pl.BlockSpec((_BLK, _D), lambda i: (i, 0)),
        out_shape=jax.ShapeDtypeStruct((_NPAD, _D), jnp.float32),
    )(z_pad, W1, b1r, Wg, deg)


def _tc_post(acc, g, deg, bgr, W2, b2r):
    def body(acc_ref, g_ref, deg_ref, bg_ref, w2_ref, b2_ref, out_ref):
        i = pl.program_id(0)
        dsum = (
            deg_ref[0, pl.ds(i * _BLK, _BLK)]
            + deg_ref[1, pl.ds(i * _BLK, _BLK)]
            + 1.0
        )
        dinv = lax.rsqrt(dsum)
        x = (acc_ref[0] + acc_ref[1] + g_ref[...]) * dinv[:, None]
        h3 = jnp.maximum(x + bg_ref[...], 0.0)
        out_ref[...] = (
            jnp.dot(h3, w2_ref[...], preferred_element_type=jnp.float32)
            + b2_ref[...]
        )

    return pl.pallas_call(
        body,
        grid=(_NPAD // _BLK,),
        in_specs=[
            pl.BlockSpec((_NC, _BLK, _D), lambda i: (0, i, 0)),
            pl.BlockSpec((_BLK, _D), lambda i: (i, 0)),
            pl.BlockSpec((_NC, _NPAD), lambda i: (0, 0)),
            pl.BlockSpec((1, _D), lambda i: (0, 0)),
            pl.BlockSpec((_D, _D), lambda i: (0, 0)),
            pl.BlockSpec((1, _D), lambda i: (0, 0)),
        ],
        out_specs=pl.BlockSpec((_BLK, _D), lambda i: (i, 0)),
        out_shape=jax.ShapeDtypeStruct((_NPAD, _D), jnp.float32),
    )(acc, g, deg, bgr, W2, b2r)


def kernel(z, edge_index, W1, b1, Wg, bg, W2, b2):
    src = edge_index[0]
    dst = edge_index[1]
    pad_e = _EP - _E
    src_p = jnp.concatenate([src, jnp.zeros((pad_e,), jnp.int32)])
    dst_p = jnp.concatenate([dst, jnp.full((pad_e,), _N, jnp.int32)])
    z_pad = jnp.pad(z, ((0, _NPAD - _N), (0, 0)))

    deg = _sc_degree(dst_p)
    g = _tc_pre(z_pad, W1, b1.reshape(1, _D), Wg, deg)
    acc = _sc_gather_scatter(g, src_p, dst_p)
    out = _tc_post(acc, g, deg, bg.reshape(1, _D), W2, b2.reshape(1, _D))
    return out[:_N]
